# baseline (speedup 1.0000x reference)
import functools

import jax
import jax.numpy as jnp
from jax import lax
from jax.experimental import pallas as pl
from jax.experimental.pallas import tpu as pltpu, tpu_sc as plsc

NC, NS = 2, 16
NW = NC * NS


def _make_gather(V, D, B0, S):
    assert B0 % NW == 0 and D % 16 == 0 and S % 2 == 0
    n_per_w = B0 // NW
    P = S // 2

    mesh = plsc.VectorSubcoreMesh(core_axis_name="c", subcore_axis_name="s")

    @functools.partial(
        pl.kernel,
        out_type=jax.ShapeDtypeStruct((S, B0, D), jnp.float32),
        mesh=mesh,
        compiler_params=pltpu.CompilerParams(skip_device_barrier=True),
        scratch_types=[
            pltpu.VMEM((S, n_per_w), jnp.int32),
            pltpu.VMEM((2, n_per_w, D), jnp.float32),
            pltpu.VMEM((2, n_per_w, D), jnp.float32),
            pltpu.VMEM((2, n_per_w, D), jnp.float32),
            pltpu.SemaphoreType.DMA,
            pltpu.SemaphoreType.DMA,
        ],
    )
    def gather(table_hbm, tok_hbm, out_hbm, idx_v, buf0, buf1, buf2,
               sem_g, sem_s):
        bufs = (buf0, buf1, buf2)
        wid = lax.axis_index("s") * NC + lax.axis_index("c")
        col = wid * n_per_w
        pltpu.sync_copy(tok_hbm.at[:, pl.ds(col, n_per_w)], idx_v)

        def fire_gathers(p, buf):
            pltpu.async_copy(table_hbm.at[idx_v.at[2 * p]], buf.at[0],
                             sem_g)
            pltpu.async_copy(table_hbm.at[idx_v.at[2 * p + 1]], buf.at[1],
                             sem_g)

        def wait_gathers(buf):
            for h in range(2):
                pltpu.make_async_copy(table_hbm.at[idx_v.at[0]],
                                      buf.at[h], sem_g).wait()

        def fire_scatter(p, buf):
            pltpu.async_copy(buf, out_hbm.at[pl.ds(2 * p, 2),
                                             pl.ds(col, n_per_w)], sem_s)

        def wait_scatter(buf):
            pltpu.make_async_copy(buf, out_hbm.at[pl.ds(0, 2),
                                                  pl.ds(col, n_per_w)],
                                  sem_s).wait()

        NB = 3
        K = 2

        def step(p, b, wait_s, fire_n):
            wait_gathers(bufs[b])
            fire_scatter(p, bufs[b])
            if fire_n:
                if wait_s:
                    wait_scatter(bufs[(b + K) % NB])
                fire_gathers(p + K, bufs[(b + K) % NB])

        for j in range(K):
            fire_gathers(j, bufs[j])

        for p in range(NB):
            step(p, p % NB, p >= NB - K, True)

        @pl.loop(1, (P - 7) // NB + 1)
        def _(g):
            p0 = NB * g
            for b in range(NB):
                step(p0 + b, b, True, True)

        for p in range(P - 4, P):
            step(p, p % NB, p + K < P, p + K < P)

        for b in range(NB):
            wait_scatter(bufs[b])

    return gather


def kernel(tokens, table):
    B0, S = tokens.shape
    V, D = table.shape
    tok_t = jnp.transpose(tokens.astype(jnp.int32))
    out_t = _make_gather(V, D, B0, S)(table, tok_t)
    return jnp.transpose(out_t, (1, 0, 2))


# speedup vs baseline: 1.0021x; 1.0021x over previous
import functools

import jax
import jax.numpy as jnp
from jax import lax
from jax.experimental import pallas as pl
from jax.experimental.pallas import tpu as pltpu, tpu_sc as plsc

NC, NS = 2, 16
NW = NC * NS


def _make_gather(V, D, B0, S):
    assert B0 % NW == 0 and D % 16 == 0 and S % 2 == 0
    n_per_w = B0 // NW
    P = S // 2

    mesh = plsc.VectorSubcoreMesh(core_axis_name="c", subcore_axis_name="s")

    @functools.partial(
        pl.kernel,
        out_type=jax.ShapeDtypeStruct((S, B0, D), jnp.float32),
        mesh=mesh,
        scratch_types=[
            pltpu.VMEM((S, n_per_w), jnp.int32),
            pltpu.VMEM((2, n_per_w, D), jnp.float32),
            pltpu.VMEM((2, n_per_w, D), jnp.float32),
            pltpu.VMEM((2, n_per_w, D), jnp.float32),
            pltpu.SemaphoreType.DMA,
            pltpu.SemaphoreType.DMA,
        ],
    )
    def gather(table_hbm, tok_hbm, out_hbm, idx_v, buf0, buf1, buf2,
               sem_g, sem_s):
        bufs = (buf0, buf1, buf2)
        wid = lax.axis_index("s") * NC + lax.axis_index("c")
        col = wid * n_per_w
        pltpu.sync_copy(tok_hbm.at[:, pl.ds(col, n_per_w)], idx_v)

        def fire_gathers(p, buf):
            pltpu.async_copy(table_hbm.at[idx_v.at[2 * p]], buf.at[0],
                             sem_g)
            pltpu.async_copy(table_hbm.at[idx_v.at[2 * p + 1]], buf.at[1],
                             sem_g)

        def wait_gathers(buf):
            for h in range(2):
                pltpu.make_async_copy(table_hbm.at[idx_v.at[0]],
                                      buf.at[h], sem_g).wait()

        def fire_scatter(p, buf):
            pltpu.async_copy(buf, out_hbm.at[pl.ds(2 * p, 2),
                                             pl.ds(col, n_per_w)], sem_s)

        def wait_scatter(buf):
            pltpu.make_async_copy(buf, out_hbm.at[pl.ds(0, 2),
                                                  pl.ds(col, n_per_w)],
                                  sem_s).wait()

        NB = 3
        K = 2

        def step(p, b, wait_s, fire_n):
            wait_gathers(bufs[b])
            fire_scatter(p, bufs[b])
            if fire_n:
                if wait_s:
                    wait_scatter(bufs[(b + K) % NB])
                fire_gathers(p + K, bufs[(b + K) % NB])

        for j in range(K):
            fire_gathers(j, bufs[j])

        for p in range(NB):
            step(p, p % NB, p >= NB - K, True)

        @pl.loop(1, (P - 7) // NB + 1)
        def _(g):
            p0 = NB * g
            for b in range(NB):
                step(p0 + b, b, True, True)

        for p in range(P - 4, P):
            step(p, p % NB, p + K < P, p + K < P)

        for b in range(NB):
            wait_scatter(bufs[b])

    return gather


def kernel(tokens, table):
    B0, S = tokens.shape
    V, D = table.shape
    tok_t = jnp.transpose(tokens.astype(jnp.int32))
    out_t = _make_gather(V, D, B0, S)(table, tok_t)
    return jnp.transpose(out_t, (1, 0, 2))
